# Initial kernel scaffold; baseline (speedup 1.0000x reference)
import jax
import jax.numpy as jnp
from jax.experimental import pallas as pl


def kernel(V_set, E_set, node_ids):
    raise NotImplementedError("write your pallas kernel here")


# speedup vs baseline: 1.5350x; 1.5350x over previous
import functools

import jax
import jax.numpy as jnp
from jax import lax
from jax.experimental import pallas as pl
from jax.experimental.pallas import tpu as pltpu
from jax.experimental.pallas import tpu_sc as plsc

_NUM_WORKERS = 32
_CHUNK = 400


def kernel(V_set, E_set, node_ids):
    V = V_set[0]
    E = E_set[0]
    idx = node_ids[0].astype(jnp.int32)
    M, De = E.shape
    D = V.shape[1]
    b_per_w = M // _NUM_WORKERS
    n_chunks = b_per_w // _CHUNK

    mesh = plsc.VectorSubcoreMesh(core_axis_name="c", subcore_axis_name="s")

    @functools.partial(
        pl.kernel,
        mesh=mesh,
        out_type=jax.ShapeDtypeStruct((M, De + D), jnp.float32),
        scratch_types=[
            pltpu.VMEM((_CHUNK,), jnp.int32),
            pltpu.VMEM((_CHUNK, D), jnp.float32),
            pltpu.VMEM((_CHUNK, De), jnp.float32),
            pltpu.SemaphoreType.DMA,
        ],
        compiler_params=pltpu.CompilerParams(use_tc_tiling_on_sc=False),
    )
    def _k(v_hbm, e_hbm, idx_hbm, out_hbm, idx_v, rows_v, e_v, sem):
        wid = lax.axis_index("s") * 2 + lax.axis_index("c")
        base = wid * b_per_w

        def body(c, carry):
            off = base + c * _CHUNK
            pltpu.sync_copy(idx_hbm.at[pl.ds(off, _CHUNK)], idx_v)
            pltpu.async_copy(v_hbm.at[idx_v], rows_v, sem).wait()
            pltpu.sync_copy(e_hbm.at[pl.ds(off, _CHUNK)], e_v)
            pltpu.sync_copy(e_v, out_hbm.at[pl.ds(off, _CHUNK), pl.ds(0, De)])
            pltpu.sync_copy(rows_v, out_hbm.at[pl.ds(off, _CHUNK), pl.ds(De, D)])
            return carry

        lax.fori_loop(0, n_chunks, body, 0)

    out = _k(V, E, idx)
    return out[jnp.newaxis]
